# Initial kernel scaffold; baseline (speedup 1.0000x reference)
import jax
import jax.numpy as jnp
from jax.experimental import pallas as pl


def kernel(x, edge_index, edge_attr, W_z, b_z, W_r, b_r, W_h, b_h, lz_W, lz_b, lr_W, lr_b, lh_W, lh_b, l1_W, l1_b, l2_W, l2_b):
    raise NotImplementedError("write your pallas kernel here")


# speedup vs baseline: 2.1947x; 2.1947x over previous
import functools

import jax
import jax.numpy as jnp
from jax.experimental import pallas as pl
from jax.experimental.pallas import tpu as pltpu

N = 10000
E = 320000
NF = 128
OC = 128
LD = 256
T_IN = 12
T_OUT = 12

BLK = 512
NPAD = 10240


def _gru_head_body(ax_ref, xt_ref, dv2_ref,
                   Wz_ref, bz_ref, Wr_ref, br_ref, Wh_ref, bh_ref,
                   lzW_ref, lzb_ref, lrW_ref, lrb_ref, lhW_ref, lhb_ref,
                   l1W_ref, l1b_ref, l2W_ref, l2b_ref, out_ref):
    f32 = jnp.float32
    dot = functools.partial(jnp.dot, preferred_element_type=f32)
    dv2 = dv2_ref[:]
    H = jnp.zeros((BLK, OC), f32)
    for t in range(T_IN):
        C = ax_ref[t] + dv2 * xt_ref[t]
        Gz = dot(C, Wz_ref[:]) + bz_ref[:]
        Gr = dot(C, Wr_ref[:]) + br_ref[:]
        Gh = dot(C, Wh_ref[:]) + bh_ref[:]
        Z = jax.nn.sigmoid(dot(Gz, lzW_ref[:OC]) + dot(H, lzW_ref[OC:]) + lzb_ref[:])
        R = jax.nn.sigmoid(dot(Gr, lrW_ref[:OC]) + dot(H, lrW_ref[OC:]) + lrb_ref[:])
        Ht = jnp.tanh(dot(Gh, lhW_ref[:OC]) + dot(H * R, lhW_ref[OC:]) + lhb_ref[:])
        H = Z * H + (1.0 - Z) * Ht
    h = jax.nn.relu(H)
    h = jax.nn.relu(dot(h, l1W_ref[:]) + l1b_ref[:])
    out_ref[:] = dot(h, l2W_ref[:]) + l2b_ref[:]


def _gru_head(ax, xt, dv2, Wz, bz, Wr, br, Wh, bh,
              lzW, lzb, lrW, lrb, lhW, lhb, l1W, l1b, l2W, l2b):
    grid = NPAD // BLK
    full = lambda shape: pl.BlockSpec(shape, lambda i: (0,) * len(shape))
    return pl.pallas_call(
        _gru_head_body,
        grid=(grid,),
        in_specs=[
            pl.BlockSpec((T_IN, BLK, NF), lambda i: (0, i, 0)),
            pl.BlockSpec((T_IN, BLK, NF), lambda i: (0, i, 0)),
            pl.BlockSpec((BLK, 1), lambda i: (i, 0)),
            full((NF, OC)), full((1, OC)),
            full((NF, OC)), full((1, OC)),
            full((NF, OC)), full((1, OC)),
            full((2 * OC, OC)), full((1, OC)),
            full((2 * OC, OC)), full((1, OC)),
            full((2 * OC, OC)), full((1, OC)),
            full((OC, LD)), full((1, LD)),
            full((LD, T_OUT)), full((1, T_OUT)),
        ],
        out_specs=pl.BlockSpec((BLK, T_OUT), lambda i: (i, 0)),
        out_shape=jax.ShapeDtypeStruct((NPAD, T_OUT), jnp.float32),
        compiler_params=pltpu.CompilerParams(
            dimension_semantics=("arbitrary",),
        ),
    )(ax, xt, dv2, Wz, bz, Wr, br, Wh, bh,
      lzW, lzb, lrW, lrb, lhW, lhb, l1W, l1b, l2W, l2b)


def kernel(x, edge_index, edge_attr, W_z, b_z, W_r, b_r, W_h, b_h,
           lz_W, lz_b, lr_W, lr_b, lh_W, lh_b, l1_W, l1_b, l2_W, l2_b):
    src, dst = edge_index[0], edge_index[1]
    ew = edge_attr

    deg = jnp.zeros((N,), jnp.float32).at[dst].add(ew) + 1.0
    dinv = deg ** -0.5
    norm = dinv[src] * ew * dinv[dst]
    xT = jnp.transpose(x, (2, 0, 1))
    ax = []
    for t in range(T_IN):
        g = xT[t][src] * norm[:, None]
        ax.append(jax.ops.segment_sum(g, dst, num_segments=N))
    AX = jnp.stack(ax, axis=0)

    pad = NPAD - N
    AXp = jnp.pad(AX, ((0, 0), (0, pad), (0, 0)))
    xTp = jnp.pad(xT, ((0, 0), (0, pad), (0, 0)))
    dv2 = jnp.pad((dinv * dinv)[:, None], ((0, pad), (0, 0)))
    r2 = lambda v: v.reshape(1, -1)
    out = _gru_head(AXp, xTp, dv2,
                    W_z, r2(b_z), W_r, r2(b_r), W_h, r2(b_h),
                    lz_W, r2(lz_b), lr_W, r2(lr_b), lh_W, r2(lh_b),
                    l1_W, r2(l1_b), l2_W, r2(l2_b))
    return out[:N]


# speedup vs baseline: 11.1941x; 5.1006x over previous
import functools

import jax
import jax.numpy as jnp
from jax import lax
from jax.experimental import pallas as pl
from jax.experimental.pallas import tpu as pltpu
from jax.experimental.pallas import tpu_sc as plsc

N = 10000
E = 320000
NF = 128
OC = 128
LD = 256
T_IN = 12
T_OUT = 12

NSC = 2
NTILE = 16
TILE_E = E // NTILE
BATCH = 128
NPADR = 10240
RSLICE = NPADR // NTILE
OSLICE = 632
OTAIL = N - (NTILE - 1) * OSLICE
NHALF = N // NSC
ACCR = 5120
EBUF = TILE_E + BATCH
WSL = 312
WTAIL = NHALF - (NTILE - 1) * WSL

_BLK = 400


def _rsqrt_newton(v):
    xi = lax.bitcast_convert_type(v, jnp.int32)
    y = lax.bitcast_convert_type(jnp.int32(0x5F3759DF) - (xi >> 1), jnp.float32)
    for _ in range(3):
        y = y * (1.5 - 0.5 * v * y * y)
    return y


def _sc_body(xflat_h, src_h, dst_h, ew_h, ones_h,
             ax_h, dv2_h,
             src_v, dst_v, ew_v, dinv_v, slice_v, d2_v, rows_v,
             gidx_v, sidx_v, deg_s, acc_s, sem):
    c = lax.axis_index("c")
    s = lax.axis_index("s")
    ebase = s * TILE_E
    rbase = s * RSLICE

    pltpu.sync_copy(src_h.at[pl.ds(ebase, TILE_E)], src_v.at[pl.ds(0, TILE_E)])
    pltpu.sync_copy(dst_h.at[pl.ds(ebase, TILE_E)], dst_v.at[pl.ds(0, TILE_E)])
    pltpu.sync_copy(ew_h.at[pl.ds(ebase, TILE_E)], ew_v.at[pl.ds(0, TILE_E)])

    gtail = jnp.full((16,), N, jnp.int32)
    ztail = jnp.zeros((16,), jnp.float32)
    for j in range(BATCH // 16):
        tsl = pl.ds(TILE_E + j * 16, 16)
        dst_v[tsl] = gtail
        ew_v[tsl] = ztail

    pltpu.sync_copy(ones_h.at[pl.ds(0, RSLICE)], slice_v)
    pltpu.sync_copy(slice_v, deg_s.at[pl.ds(rbase, RSLICE)])
    plsc.subcore_barrier()
    pltpu.sync_copy(ew_v, deg_s.at[dst_v], add=True)
    plsc.subcore_barrier()

    pltpu.sync_copy(deg_s.at[pl.ds(rbase, RSLICE)], slice_v)

    def dinv_body(j, _):
        sl = pl.ds(j * 16, 16)
        y = _rsqrt_newton(slice_v[sl])
        slice_v[sl] = y
        d2_v[sl] = y * y
        return 0

    lax.fori_loop(0, RSLICE // 16, dinv_body, 0)
    pltpu.sync_copy(slice_v, deg_s.at[pl.ds(rbase, RSLICE)])

    @pl.when(c == 0)
    def _():
        pltpu.sync_copy(d2_v, dv2_h.at[pl.ds(rbase, RSLICE)])

    plsc.subcore_barrier()

    pltpu.sync_copy(deg_s, dinv_v)

    def norm_body(j, _):
        sl = pl.ds(j * 16, 16)
        a = plsc.load_gather(dinv_v, [src_v[sl]])
        b = plsc.load_gather(dinv_v, [dst_v[sl]])
        ew_v[sl] = a * ew_v[sl] * b
        return 0

    lax.fori_loop(0, TILE_E // 16, norm_body, 0)

    lo = c * NHALF

    def part_body(j, carry):
        sl = pl.ds(j * 16, 16)
        s16 = src_v[sl]
        d16 = dst_v[sl] - lo
        n16 = ew_v[sl]
        m = (d16 >= 0) & (d16 < NHALF)
        wsl = pl.ds(carry, 16)
        plsc.store_compressed(src_v.at[wsl], s16, mask=m)
        plsc.store_compressed(dst_v.at[wsl], d16, mask=m)
        plsc.store_compressed(ew_v.at[wsl], n16, mask=m)
        return carry + jnp.sum(m.astype(jnp.int32))

    cnt = lax.fori_loop(0, TILE_E // 16, part_body, 0)
    nb = (cnt + BATCH - 1) >> 7
    gz = jnp.zeros((16,), jnp.float32)
    gi = jnp.zeros((16,), jnp.int32)
    gg = jnp.full((16,), NHALF, jnp.int32) + lax.iota(jnp.int32, 16)
    for j in range(BATCH // 16):
        psl = pl.ds(cnt + j * 16, 16)
        src_v[psl] = gi
        dst_v[psl] = gg
        ew_v[psl] = gz

    def chunk_body(t, _):
        goff = t * N
        ooff = t * N + lo

        z16 = jnp.zeros((16,), jnp.float32)

        def zrow_body(r, _):
            for v in range(NF // 16):
                rows_v[r, pl.ds(v * 16, 16)] = z16
            return 0

        lax.fori_loop(0, BATCH, zrow_body, 0)
        abase = s * (ACCR // NTILE)
        pltpu.sync_copy(rows_v, acc_s.at[pl.ds(abase, BATCH)])
        pltpu.sync_copy(rows_v, acc_s.at[pl.ds(abase + BATCH, BATCH)])
        pltpu.sync_copy(rows_v.at[pl.ds(0, 64)],
                        acc_s.at[pl.ds(abase + 2 * BATCH, 64)])
        plsc.subcore_barrier()

        def batch_body(b, _):
            eb = b * BATCH

            def idx_body(j, _):
                sl = pl.ds(j * 16, 16)
                esl = pl.ds(eb + j * 16, 16)
                gidx_v[sl] = src_v[esl] + goff
                sidx_v[sl] = dst_v[esl]
                return 0

            lax.fori_loop(0, BATCH // 16, idx_body, 0)
            pltpu.async_copy(xflat_h.at[gidx_v], rows_v, sem).wait()

            def scale_body(j, _):
                for u in range(8):
                    e = j * 8 + u
                    nv = plsc.load_gather(
                        ew_v, [jnp.full((16,), eb + e, jnp.int32)])
                    for v in range(NF // 16):
                        sl = pl.ds(v * 16, 16)
                        rows_v[e, sl] = rows_v[e, sl] * nv
                return 0

            lax.fori_loop(0, BATCH // 8, scale_body, 0)
            pltpu.sync_copy(rows_v, acc_s.at[sidx_v], add=True)
            return 0

        lax.fori_loop(0, nb, batch_body, 0)
        plsc.subcore_barrier()
        wbase = s * WSL

        @pl.when(s < NTILE - 1)
        def _():
            pltpu.sync_copy(acc_s.at[pl.ds(wbase, WSL)],
                            ax_h.at[pl.ds(ooff + wbase, WSL)])

        @pl.when(s == NTILE - 1)
        def _():
            pltpu.sync_copy(acc_s.at[pl.ds((NTILE - 1) * WSL, WTAIL)],
                            ax_h.at[pl.ds(ooff + (NTILE - 1) * WSL, WTAIL)])

        plsc.subcore_barrier()
        return 0

    lax.fori_loop(0, T_IN, chunk_body, 0)


def _spmm_sc(xflat, src, dst, ew, ones_a):
    mesh = plsc.VectorSubcoreMesh(core_axis_name="c", subcore_axis_name="s")
    f = pl.kernel(
        _sc_body,
        out_type=[
            jax.ShapeDtypeStruct((T_IN * N, NF), jnp.float32),
            jax.ShapeDtypeStruct((NPADR,), jnp.float32),
        ],
        mesh=mesh,
        compiler_params=pltpu.CompilerParams(needs_layout_passes=False),
        scratch_types=[
            pltpu.VMEM((EBUF,), jnp.int32),
            pltpu.VMEM((EBUF,), jnp.int32),
            pltpu.VMEM((EBUF,), jnp.float32),
            pltpu.VMEM((NPADR,), jnp.float32),
            pltpu.VMEM((RSLICE,), jnp.float32),
            pltpu.VMEM((RSLICE,), jnp.float32),
            pltpu.VMEM((BATCH, NF), jnp.float32),
            pltpu.VMEM((BATCH,), jnp.int32),
            pltpu.VMEM((BATCH,), jnp.int32),
            pltpu.VMEM_SHARED((NPADR,), jnp.float32),
            pltpu.VMEM_SHARED((ACCR, NF), jnp.float32),
            pltpu.SemaphoreType.DMA,
        ],
    )
    return f(xflat, src, dst, ew, ones_a)


def _gru_head_body(ax_ref, xt_ref, dv2_ref,
                   Wz_ref, bz_ref, Wr_ref, br_ref, Wh_ref, bh_ref,
                   lzW_ref, lzb_ref, lrW_ref, lrb_ref, lhW_ref, lhb_ref,
                   l1W_ref, l1b_ref, l2W_ref, l2b_ref, out_ref):
    f32 = jnp.float32
    dot = functools.partial(jnp.dot, preferred_element_type=f32)
    dv2 = dv2_ref[:]
    H = jnp.zeros((_BLK, OC), f32)
    for t in range(T_IN):
        C = ax_ref[t] + dv2 * xt_ref[t]
        Gz = dot(C, Wz_ref[:]) + bz_ref[:]
        Gr = dot(C, Wr_ref[:]) + br_ref[:]
        Gh = dot(C, Wh_ref[:]) + bh_ref[:]
        Z = jax.nn.sigmoid(dot(Gz, lzW_ref[:OC]) + dot(H, lzW_ref[OC:]) + lzb_ref[:])
        R = jax.nn.sigmoid(dot(Gr, lrW_ref[:OC]) + dot(H, lrW_ref[OC:]) + lrb_ref[:])
        Ht = jnp.tanh(dot(Gh, lhW_ref[:OC]) + dot(H * R, lhW_ref[OC:]) + lhb_ref[:])
        H = Z * H + (1.0 - Z) * Ht
    h = jax.nn.relu(H)
    h = jax.nn.relu(dot(h, l1W_ref[:]) + l1b_ref[:])
    out_ref[:] = dot(h, l2W_ref[:]) + l2b_ref[:]


def _gru_head(ax, xt, dv2, Wz, bz, Wr, br, Wh, bh,
              lzW, lzb, lrW, lrb, lhW, lhb, l1W, l1b, l2W, l2b):
    grid = N // _BLK
    full = lambda shape: pl.BlockSpec(shape, lambda i: (0,) * len(shape))
    return pl.pallas_call(
        _gru_head_body,
        grid=(grid,),
        in_specs=[
            pl.BlockSpec((T_IN, _BLK, NF), lambda i: (0, i, 0)),
            pl.BlockSpec((T_IN, _BLK, NF), lambda i: (0, i, 0)),
            pl.BlockSpec((_BLK, 1), lambda i: (i, 0)),
            full((NF, OC)), full((1, OC)),
            full((NF, OC)), full((1, OC)),
            full((NF, OC)), full((1, OC)),
            full((2 * OC, OC)), full((1, OC)),
            full((2 * OC, OC)), full((1, OC)),
            full((2 * OC, OC)), full((1, OC)),
            full((OC, LD)), full((1, LD)),
            full((LD, T_OUT)), full((1, T_OUT)),
        ],
        out_specs=pl.BlockSpec((_BLK, T_OUT), lambda i: (i, 0)),
        out_shape=jax.ShapeDtypeStruct((N, T_OUT), jnp.float32),
        compiler_params=pltpu.CompilerParams(
            dimension_semantics=("arbitrary",),
        ),
    )(ax, xt, dv2, Wz, bz, Wr, br, Wh, bh,
      lzW, lzb, lrW, lrb, lhW, lhb, l1W, l1b, l2W, l2b)


def kernel(x, edge_index, edge_attr, W_z, b_z, W_r, b_r, W_h, b_h,
           lz_W, lz_b, lr_W, lr_b, lh_W, lh_b, l1_W, l1_b, l2_W, l2_b):
    src = edge_index[0].astype(jnp.int32)
    dst = edge_index[1].astype(jnp.int32)
    ew = edge_attr

    xT = jnp.transpose(x, (2, 0, 1))
    xflat = xT.reshape(T_IN * N, NF)
    ones_a = jnp.ones((RSLICE,), jnp.float32)

    axflat, dv2 = _spmm_sc(xflat, src, dst, ew, ones_a)
    AX = axflat.reshape(T_IN, N, NF)
    dv2 = dv2[:N].reshape(N, 1)

    r2 = lambda v: v.reshape(1, -1)
    return _gru_head(AX, xT, dv2,
                     W_z, r2(b_z), W_r, r2(b_r), W_h, r2(b_h),
                     lz_W, r2(lz_b), lr_W, r2(lr_b), lh_W, r2(lh_b),
                     l1_W, r2(l1_b), l2_W, r2(l2_b))


# speedup vs baseline: 14.1078x; 1.2603x over previous
import functools

import jax
import jax.numpy as jnp
from jax import lax
from jax.experimental import pallas as pl
from jax.experimental.pallas import tpu as pltpu
from jax.experimental.pallas import tpu_sc as plsc

N = 10000
E = 320000
NF = 128
OC = 128
LD = 256
T_IN = 12
T_OUT = 12

NSC = 2
NTILE = 16
TILE_E = E // NTILE
BATCH = 128
SB = 32
NPADR = 10240
RSLICE = NPADR // NTILE
OSLICE = 632
OTAIL = N - (NTILE - 1) * OSLICE
NHALF = N // NSC
ACCR = 5120
EBUF = TILE_E + BATCH
WSL = 312
WTAIL = NHALF - (NTILE - 1) * WSL

_BLK = 400


def _rsqrt_newton(v):
    xi = lax.bitcast_convert_type(v, jnp.int32)
    y = lax.bitcast_convert_type(jnp.int32(0x5F3759DF) - (xi >> 1), jnp.float32)
    for _ in range(3):
        y = y * (1.5 - 0.5 * v * y * y)
    return y


def _sc_body(xflat_h, src_h, dst_h, ew_h, ones_h,
             ax_h, dv2_h,
             src_v, dst_v, ew_v, dinv_v, slice_v, d2_v,
             rows0, rows1, rows2, rows3,
             deg_s, acc_s,
             gsem0, gsem1, gsem2, gsem3, ssem0, ssem1, ssem2, ssem3):
    c = lax.axis_index("c")
    s = lax.axis_index("s")
    ebase = s * TILE_E
    rbase = s * RSLICE

    pltpu.sync_copy(src_h.at[pl.ds(ebase, TILE_E)], src_v.at[pl.ds(0, TILE_E)])
    pltpu.sync_copy(dst_h.at[pl.ds(ebase, TILE_E)], dst_v.at[pl.ds(0, TILE_E)])
    pltpu.sync_copy(ew_h.at[pl.ds(ebase, TILE_E)], ew_v.at[pl.ds(0, TILE_E)])

    gtail = jnp.full((16,), N, jnp.int32)
    ztail = jnp.zeros((16,), jnp.float32)
    for j in range(BATCH // 16):
        tsl = pl.ds(TILE_E + j * 16, 16)
        dst_v[tsl] = gtail
        ew_v[tsl] = ztail

    pltpu.sync_copy(ones_h.at[pl.ds(0, RSLICE)], slice_v)
    pltpu.sync_copy(slice_v, deg_s.at[pl.ds(rbase, RSLICE)])
    plsc.subcore_barrier()
    pltpu.sync_copy(ew_v, deg_s.at[dst_v], add=True)
    plsc.subcore_barrier()

    pltpu.sync_copy(deg_s.at[pl.ds(rbase, RSLICE)], slice_v)

    def dinv_body(j, _):
        sl = pl.ds(j * 16, 16)
        y = _rsqrt_newton(slice_v[sl])
        slice_v[sl] = y
        d2_v[sl] = y * y
        return 0

    lax.fori_loop(0, RSLICE // 16, dinv_body, 0)
    pltpu.sync_copy(slice_v, deg_s.at[pl.ds(rbase, RSLICE)])

    @pl.when(c == 0)
    def _():
        pltpu.sync_copy(d2_v, dv2_h.at[pl.ds(rbase, RSLICE)])

    plsc.subcore_barrier()

    pltpu.sync_copy(deg_s, dinv_v)

    def norm_body(j, _):
        sl = pl.ds(j * 16, 16)
        a = plsc.load_gather(dinv_v, [src_v[sl]])
        b = plsc.load_gather(dinv_v, [dst_v[sl]])
        ew_v[sl] = a * ew_v[sl] * b
        return 0

    lax.fori_loop(0, TILE_E // 16, norm_body, 0)

    lo = c * NHALF

    def part_body(j, carry):
        sl = pl.ds(j * 16, 16)
        s16 = src_v[sl]
        d16 = dst_v[sl] - lo
        n16 = ew_v[sl]
        m = (d16 >= 0) & (d16 < NHALF)
        wsl = pl.ds(carry, 16)
        plsc.store_compressed(src_v.at[wsl], s16, mask=m)
        plsc.store_compressed(dst_v.at[wsl], d16, mask=m)
        plsc.store_compressed(ew_v.at[wsl], n16, mask=m)
        return carry + jnp.sum(m.astype(jnp.int32))

    cnt = lax.fori_loop(0, TILE_E // 16, part_body, 0)
    nq = jnp.maximum((cnt + 4 * SB - 1) // (4 * SB), 1)
    gz = jnp.zeros((16,), jnp.float32)
    gi = jnp.zeros((16,), jnp.int32)
    gg = jnp.full((16,), NHALF, jnp.int32) + lax.iota(jnp.int32, 16)
    for j in range(4 * SB // 16):
        psl = pl.ds(cnt + j * 16, 16)
        src_v[psl] = gi
        dst_v[psl] = gg
        ew_v[psl] = gz

    rows = [rows0, rows1, rows2, rows3]
    gsem = [gsem0, gsem1, gsem2, gsem3]
    ssem = [ssem0, ssem1, ssem2, ssem3]

    def gather_start(k, j):
        pltpu.async_copy(xflat_h.at[src_v.at[pl.ds(k * SB, SB)]],
                         rows[j], gsem[j])

    def gather_wait(k, j):
        pltpu.make_async_copy(xflat_h.at[src_v.at[pl.ds(k * SB, SB)]],
                              rows[j], gsem[j]).wait()

    def scatter_start(k, j):
        pltpu.async_copy(rows[j], acc_s.at[dst_v.at[pl.ds(k * SB, SB)]],
                         ssem[j], add=True)

    def scatter_wait(k, j):
        pltpu.make_async_copy(rows[j], acc_s.at[dst_v.at[pl.ds(k * SB, SB)]],
                              ssem[j]).wait()

    def chunk_body(t, _):
        ooff = t * N + lo

        @pl.when(t > 0)
        def _():
            def bump_body(i, _):
                sl = pl.ds(i * 16, 16)
                src_v[sl] = src_v[sl] + N
                return 0
            lax.fori_loop(0, EBUF // 16, bump_body, 0)

        z16 = jnp.zeros((16,), jnp.float32)

        def zrow_body(r, _):
            for v in range(NF // 16):
                rows0[r, pl.ds(v * 16, 16)] = z16
            return 0

        lax.fori_loop(0, SB, zrow_body, 0)
        abase = s * (ACCR // NTILE)
        for r in range(ACCR // NTILE // SB):
            pltpu.sync_copy(rows0, acc_s.at[pl.ds(abase + r * SB, SB)])
        plsc.subcore_barrier()

        gather_start(0, 0)
        gather_start(1, 1)

        def quad_body(q, _):
            for j in range(4):
                k = 4 * q + j
                gather_wait(k, j)
                eb = k * SB

                def scale_body(j2, _):
                    for u in range(8):
                        e = j2 * 8 + u
                        nv = plsc.load_gather(
                            ew_v, [jnp.full((16,), eb + e, jnp.int32)])
                        for v in range(NF // 16):
                            sl = pl.ds(v * 16, 16)
                            rows[j][e, sl] = rows[j][e, sl] * nv
                    return 0

                lax.fori_loop(0, SB // 8, scale_body, 0)
                scatter_start(k, j)
                jb = (j + 2) % 4
                if j < 2:
                    @pl.when(q >= 1)
                    def _():
                        scatter_wait(k - 2, jb)
                    gather_start(k + 2, jb)
                else:
                    @pl.when(q + 1 < nq)
                    def _():
                        scatter_wait(k - 2, jb)
                        gather_start(k + 2, jb)
            return 0

        lax.fori_loop(0, nq, quad_body, 0)
        for j in range(4):
            scatter_wait(4 * (nq - 1) + j, j)
        plsc.subcore_barrier()
        wbase = s * WSL

        @pl.when(s < NTILE - 1)
        def _():
            pltpu.sync_copy(acc_s.at[pl.ds(wbase, WSL)],
                            ax_h.at[pl.ds(ooff + wbase, WSL)])

        @pl.when(s == NTILE - 1)
        def _():
            pltpu.sync_copy(acc_s.at[pl.ds((NTILE - 1) * WSL, WTAIL)],
                            ax_h.at[pl.ds(ooff + (NTILE - 1) * WSL, WTAIL)])

        plsc.subcore_barrier()
        return 0

    lax.fori_loop(0, T_IN, chunk_body, 0)


def _spmm_sc(xflat, src, dst, ew, ones_a):
    mesh = plsc.VectorSubcoreMesh(core_axis_name="c", subcore_axis_name="s")
    f = pl.kernel(
        _sc_body,
        out_type=[
            jax.ShapeDtypeStruct((T_IN * N, NF), jnp.float32),
            jax.ShapeDtypeStruct((NPADR,), jnp.float32),
        ],
        mesh=mesh,
        compiler_params=pltpu.CompilerParams(needs_layout_passes=False),
        scratch_types=[
            pltpu.VMEM((EBUF,), jnp.int32),
            pltpu.VMEM((EBUF,), jnp.int32),
            pltpu.VMEM((EBUF,), jnp.float32),
            pltpu.VMEM((NPADR,), jnp.float32),
            pltpu.VMEM((RSLICE,), jnp.float32),
            pltpu.VMEM((RSLICE,), jnp.float32),
            pltpu.VMEM((SB, NF), jnp.float32),
            pltpu.VMEM((SB, NF), jnp.float32),
            pltpu.VMEM((SB, NF), jnp.float32),
            pltpu.VMEM((SB, NF), jnp.float32),
            pltpu.VMEM_SHARED((NPADR,), jnp.float32),
            pltpu.VMEM_SHARED((ACCR, NF), jnp.float32),
            pltpu.SemaphoreType.DMA,
            pltpu.SemaphoreType.DMA,
            pltpu.SemaphoreType.DMA,
            pltpu.SemaphoreType.DMA,
            pltpu.SemaphoreType.DMA,
            pltpu.SemaphoreType.DMA,
            pltpu.SemaphoreType.DMA,
            pltpu.SemaphoreType.DMA,
        ],
    )
    return f(xflat, src, dst, ew, ones_a)


def _gru_head_body(ax_ref, xt_ref, dv2_ref,
                   Wz_ref, bz_ref, Wr_ref, br_ref, Wh_ref, bh_ref,
                   lzW_ref, lzb_ref, lrW_ref, lrb_ref, lhW_ref, lhb_ref,
                   l1W_ref, l1b_ref, l2W_ref, l2b_ref, out_ref):
    f32 = jnp.float32
    dot = functools.partial(jnp.dot, preferred_element_type=f32)
    dv2 = dv2_ref[:]
    H = jnp.zeros((_BLK, OC), f32)
    for t in range(T_IN):
        C = ax_ref[t] + dv2 * xt_ref[t]
        Gz = dot(C, Wz_ref[:]) + bz_ref[:]
        Gr = dot(C, Wr_ref[:]) + br_ref[:]
        Gh = dot(C, Wh_ref[:]) + bh_ref[:]
        Z = jax.nn.sigmoid(dot(Gz, lzW_ref[:OC]) + dot(H, lzW_ref[OC:]) + lzb_ref[:])
        R = jax.nn.sigmoid(dot(Gr, lrW_ref[:OC]) + dot(H, lrW_ref[OC:]) + lrb_ref[:])
        Ht = jnp.tanh(dot(Gh, lhW_ref[:OC]) + dot(H * R, lhW_ref[OC:]) + lhb_ref[:])
        H = Z * H + (1.0 - Z) * Ht
    h = jax.nn.relu(H)
    h = jax.nn.relu(dot(h, l1W_ref[:]) + l1b_ref[:])
    out_ref[:] = dot(h, l2W_ref[:]) + l2b_ref[:]


def _gru_head(ax, xt, dv2, Wz, bz, Wr, br, Wh, bh,
              lzW, lzb, lrW, lrb, lhW, lhb, l1W, l1b, l2W, l2b):
    grid = N // _BLK
    full = lambda shape: pl.BlockSpec(shape, lambda i: (0,) * len(shape))
    return pl.pallas_call(
        _gru_head_body,
        grid=(grid,),
        in_specs=[
            pl.BlockSpec((T_IN, _BLK, NF), lambda i: (0, i, 0)),
            pl.BlockSpec((T_IN, _BLK, NF), lambda i: (0, i, 0)),
            pl.BlockSpec((_BLK, 1), lambda i: (i, 0)),
            full((NF, OC)), full((1, OC)),
            full((NF, OC)), full((1, OC)),
            full((NF, OC)), full((1, OC)),
            full((2 * OC, OC)), full((1, OC)),
            full((2 * OC, OC)), full((1, OC)),
            full((2 * OC, OC)), full((1, OC)),
            full((OC, LD)), full((1, LD)),
            full((LD, T_OUT)), full((1, T_OUT)),
        ],
        out_specs=pl.BlockSpec((_BLK, T_OUT), lambda i: (i, 0)),
        out_shape=jax.ShapeDtypeStruct((N, T_OUT), jnp.float32),
        compiler_params=pltpu.CompilerParams(
            dimension_semantics=("arbitrary",),
        ),
    )(ax, xt, dv2, Wz, bz, Wr, br, Wh, bh,
      lzW, lzb, lrW, lrb, lhW, lhb, l1W, l1b, l2W, l2b)


def kernel(x, edge_index, edge_attr, W_z, b_z, W_r, b_r, W_h, b_h,
           lz_W, lz_b, lr_W, lr_b, lh_W, lh_b, l1_W, l1_b, l2_W, l2_b):
    src = edge_index[0].astype(jnp.int32)
    dst = edge_index[1].astype(jnp.int32)
    ew = edge_attr

    xT = jnp.transpose(x, (2, 0, 1))
    xflat = xT.reshape(T_IN * N, NF)
    ones_a = jnp.ones((RSLICE,), jnp.float32)

    axflat, dv2 = _spmm_sc(xflat, src, dst, ew, ones_a)
    AX = axflat.reshape(T_IN, N, NF)
    dv2 = dv2[:N].reshape(N, 1)

    r2 = lambda v: v.reshape(1, -1)
    return _gru_head(AX, xT, dv2,
                     W_z, r2(b_z), W_r, r2(b_r), W_h, r2(b_h),
                     lz_W, r2(lz_b), lr_W, r2(lr_b), lh_W, r2(lh_b),
                     l1_W, r2(l1_b), l2_W, r2(l2_b))
